# baseline (speedup 1.0000x reference)
import jax
import jax.numpy as jnp
from jax.experimental import pallas as pl
from jax.experimental.pallas import tpu as pltpu

B, NP1, D = 64, 577, 768
NPATCH = NP1 - 1
NEXP = 64
DENS = NPATCH // 2
S = NPATCH - DENS
KG = S // 2
NEG = float("-inf")


def _stable_desc_ranks(v):
    n = v.shape[0]
    vj = v[:, None]
    vi = v[None, :]
    gt = (vj > vi).astype(jnp.float32)
    ioj = jax.lax.broadcasted_iota(jnp.int32, (n, n), 0)
    ioi = jax.lax.broadcasted_iota(jnp.int32, (n, n), 1)
    eq_lt = ((vj == vi) & (ioj < ioi)).astype(jnp.float32)
    return jnp.sum(gt + eq_lt, axis=0)


def _dot01(A, Bm):
    return jnp.dot(A.astype(jnp.bfloat16), Bm.astype(jnp.bfloat16),
                   preferred_element_type=jnp.float32)


def _dot2(A, X):
    hi = X.astype(jnp.bfloat16)
    lo = (X - hi.astype(jnp.float32)).astype(jnp.bfloat16)
    Ab = A.astype(jnp.bfloat16)
    return (jnp.dot(Ab, hi, preferred_element_type=jnp.float32) +
            jnp.dot(Ab, lo, preferred_element_type=jnp.float32))


def _gnn_kernel(x_ref, ca_ref, ed_ref, tok_ref, attn_ref):
    xb = x_ref[0]
    ca = ca_ref[0, 0]
    ed = ed_ref[0]

    rank = _stable_desc_ranks(ca)
    iop = jax.lax.broadcasted_iota(jnp.int32, (NPATCH, NPATCH), 0)
    P = (rank[None, :] == iop.astype(jnp.float32)).astype(jnp.float32)
    PT = (rank[:, None] == iop.T.astype(jnp.float32)).astype(jnp.float32)
    patch_x = xb[1:]
    attn_s = jnp.sum(PT * ca[:, None], axis=0)
    skip_exp = jnp.dot(P[DENS:], ed[1:],
                       preferred_element_type=jnp.float32, precision=jax.lax.Precision.HIGHEST)
    nsca = attn_s[:DENS]
    sca = attn_s[DENS:]

    norm = jnp.sqrt(jnp.sum(skip_exp * skip_exp, axis=1))
    cn = skip_exp / jnp.clip(norm, 1e-12, None)[:, None]
    cnb = cn.astype(jnp.bfloat16)
    sim = jax.lax.dot_general(cnb, cnb, (((1,), (1,)), ((), ())),
                              preferred_element_type=jnp.float32)
    ior = jax.lax.broadcasted_iota(jnp.int32, (S, S), 0)
    ioc = jax.lax.broadcasted_iota(jnp.int32, (S, S), 1)
    sim = jnp.where(ior == ioc, NEG, sim)

    m1 = jnp.max(sim, axis=1)
    i1 = jnp.min(jnp.where(sim == m1[:, None], ioc, S), axis=1)
    hit1 = ioc == i1[:, None]
    sim2 = jnp.where(hit1, NEG, sim)
    m2 = jnp.max(sim2, axis=1)
    i2 = jnp.min(jnp.where(sim2 == m2[:, None], ioc, S), axis=1)
    hit2 = ioc == i2[:, None]
    ET = hit1 | hit2
    U = (ET | ET.T).astype(jnp.float32)

    def colsum(M):
        return jnp.sum(M, axis=0)

    deg0 = colsum(U)
    C0 = U * (deg0[None, :] > deg0[:, None]).astype(jnp.float32)

    def cond(st):
        _, prev, cur = st
        return prev != cur

    def body(st):
        c, _, cur = st
        deg = colsum(c)
        new = c * (deg[None, :] > deg[:, None]).astype(jnp.float32)
        return new, cur, jnp.sum(new)

    Cf, _, _ = jax.lax.while_loop(
        cond, body, (C0, jnp.float32(-1.0), jnp.sum(C0)))

    eye = (ior == ioc).astype(jnp.float32)
    F = Cf + eye
    node_deg = jnp.sum(F, axis=1)

    r2 = _stable_desc_ranks(node_deg)
    iog = jax.lax.broadcasted_iota(jnp.int32, (KG, S), 0)
    G = (r2[None, :] == iog.astype(jnp.float32)).astype(jnp.float32)
    GFt = _dot01(G, Cf.T) + G
    M2 = _dot01(GFt, P[DENS:])
    Amat = jnp.concatenate([P[:DENS], M2], axis=0)
    tokrows = _dot2(Amat, patch_x)
    sca_sel = jnp.sum(G * sca[None, :], axis=1)

    tok_ref[0, 0:1, :] = xb[0:1]
    tok_ref[0, 1:, :] = tokrows
    attn_ref[0, 0, :DENS] = nsca
    attn_ref[0, 0, DENS:] = sca_sel


def kernel(x, cls_attn, expert_distribution):
    ca3 = cls_attn.reshape(B, 1, NPATCH)
    tok, attn = pl.pallas_call(
        _gnn_kernel,
        grid=(B,),
        in_specs=[
            pl.BlockSpec((1, NP1, D), lambda b: (b, 0, 0)),
            pl.BlockSpec((1, 1, NPATCH), lambda b: (b, 0, 0)),
            pl.BlockSpec((1, NP1, NEXP), lambda b: (b, 0, 0)),
        ],
        out_specs=[
            pl.BlockSpec((1, 1 + DENS + KG, D), lambda b: (b, 0, 0)),
            pl.BlockSpec((1, 1, DENS + KG), lambda b: (b, 0, 0)),
        ],
        out_shape=[
            jax.ShapeDtypeStruct((B, 1 + DENS + KG, D), jnp.float32),
            jax.ShapeDtypeStruct((B, 1, DENS + KG), jnp.float32),
        ],
        compiler_params=pltpu.CompilerParams(
            dimension_semantics=("parallel",)),
    )(x, ca3, expert_distribution)
    return tok, attn.reshape(B, DENS + KG)


# speedup vs baseline: 1.0364x; 1.0364x over previous
import jax
import jax.numpy as jnp
from jax.experimental import pallas as pl
from jax.experimental.pallas import tpu as pltpu

B, NP1, D = 64, 577, 768
NPATCH = NP1 - 1
NEXP = 64
DENS = NPATCH // 2
S = NPATCH - DENS
KG = S // 2
NEG = float("-inf")


def _stable_desc_ranks(v):
    n = v.shape[0]
    vj = v[:, None]
    vi = v[None, :]
    gt = (vj > vi).astype(jnp.float32)
    ioj = jax.lax.broadcasted_iota(jnp.int32, (n, n), 0)
    ioi = jax.lax.broadcasted_iota(jnp.int32, (n, n), 1)
    eq_lt = ((vj == vi) & (ioj < ioi)).astype(jnp.float32)
    return jnp.sum(gt + eq_lt, axis=0)


def _dot01(A, Bm):
    return jnp.dot(A.astype(jnp.bfloat16), Bm.astype(jnp.bfloat16),
                   preferred_element_type=jnp.float32)


def _split2(X):
    hi = X.astype(jnp.bfloat16)
    lo = (X - hi.astype(jnp.float32)).astype(jnp.bfloat16)
    return hi, lo


def _dot2(A, hi, lo):
    Ab = A.astype(jnp.bfloat16)
    return (jnp.dot(Ab, hi, preferred_element_type=jnp.float32) +
            jnp.dot(Ab, lo, preferred_element_type=jnp.float32))


def _gnn_kernel(x_ref, ca_ref, ed_ref, tok_ref, attn_ref):
    xb = x_ref[0]
    ca = ca_ref[0, 0]
    ed = ed_ref[0]

    rank = _stable_desc_ranks(ca)
    iop = jax.lax.broadcasted_iota(jnp.int32, (NPATCH, NPATCH), 0)
    P = (rank[None, :] == iop.astype(jnp.float32)).astype(jnp.float32)
    PT = (rank[:, None] == iop.T.astype(jnp.float32)).astype(jnp.float32)
    patch_x = xb[1:]
    px_hi, px_lo = _split2(patch_x)
    patch_tk = _dot2(P[:DENS], px_hi, px_lo)
    attn_s = jnp.sum(PT * ca[:, None], axis=0)
    skip_exp = jnp.dot(P[DENS:], ed[1:],
                       preferred_element_type=jnp.float32, precision=jax.lax.Precision.HIGHEST)
    nsca = attn_s[:DENS]
    sca = attn_s[DENS:]

    norm = jnp.sqrt(jnp.sum(skip_exp * skip_exp, axis=1))
    cn = skip_exp / jnp.clip(norm, 1e-12, None)[:, None]
    cnb = cn.astype(jnp.bfloat16)
    sim = jax.lax.dot_general(cnb, cnb, (((1,), (1,)), ((), ())),
                              preferred_element_type=jnp.float32)
    ior = jax.lax.broadcasted_iota(jnp.int32, (S, S), 0)
    ioc = jax.lax.broadcasted_iota(jnp.int32, (S, S), 1)
    sim = jnp.where(ior == ioc, NEG, sim)

    m1 = jnp.max(sim, axis=1)
    i1 = jnp.min(jnp.where(sim == m1[:, None], ioc, S), axis=1)
    hit1 = ioc == i1[:, None]
    sim2 = jnp.where(hit1, NEG, sim)
    m2 = jnp.max(sim2, axis=1)
    i2 = jnp.min(jnp.where(sim2 == m2[:, None], ioc, S), axis=1)
    hit2 = ioc == i2[:, None]
    ET = hit1 | hit2
    U = (ET | ET.T).astype(jnp.float32)

    def colsum(M):
        return jnp.sum(M, axis=0)

    deg0 = colsum(U)
    C0 = U * (deg0[None, :] > deg0[:, None]).astype(jnp.float32)

    def cond(st):
        _, prev, cur = st
        return prev != cur

    def body(st):
        c, _, cur = st
        deg = colsum(c)
        new = c * (deg[None, :] > deg[:, None]).astype(jnp.float32)
        return new, cur, jnp.sum(new)

    Cf, _, _ = jax.lax.while_loop(
        cond, body, (C0, jnp.float32(-1.0), jnp.sum(C0)))

    eye = (ior == ioc).astype(jnp.float32)
    F = Cf + eye
    node_deg = jnp.sum(F, axis=1)

    r2 = _stable_desc_ranks(node_deg)
    iog = jax.lax.broadcasted_iota(jnp.int32, (KG, S), 0)
    G = (r2[None, :] == iog.astype(jnp.float32)).astype(jnp.float32)
    GFt = _dot01(G, Cf.T) + G
    M2 = _dot01(GFt, P[DENS:])
    summaries = _dot2(M2, px_hi, px_lo)
    sca_sel = jnp.sum(G * sca[None, :], axis=1)

    tok_ref[0, 0:1, :] = xb[0:1]
    tok_ref[0, 1:1 + DENS, :] = patch_tk
    tok_ref[0, 1 + DENS:, :] = summaries
    attn_ref[0, 0, :DENS] = nsca
    attn_ref[0, 0, DENS:] = sca_sel


def kernel(x, cls_attn, expert_distribution):
    ca3 = cls_attn.reshape(B, 1, NPATCH)
    tok, attn = pl.pallas_call(
        _gnn_kernel,
        grid=(B,),
        in_specs=[
            pl.BlockSpec((1, NP1, D), lambda b: (b, 0, 0)),
            pl.BlockSpec((1, 1, NPATCH), lambda b: (b, 0, 0)),
            pl.BlockSpec((1, NP1, NEXP), lambda b: (b, 0, 0)),
        ],
        out_specs=[
            pl.BlockSpec((1, 1 + DENS + KG, D), lambda b: (b, 0, 0)),
            pl.BlockSpec((1, 1, DENS + KG), lambda b: (b, 0, 0)),
        ],
        out_shape=[
            jax.ShapeDtypeStruct((B, 1 + DENS + KG, D), jnp.float32),
            jax.ShapeDtypeStruct((B, 1, DENS + KG), jnp.float32),
        ],
        compiler_params=pltpu.CompilerParams(
            dimension_semantics=("parallel",)),
    )(x, ca3, expert_distribution)
    return tok, attn.reshape(B, DENS + KG)


# speedup vs baseline: 1.1462x; 1.1059x over previous
import jax
import jax.numpy as jnp
from jax.experimental import pallas as pl
from jax.experimental.pallas import tpu as pltpu

B, NP1, D = 64, 577, 768
NPATCH = NP1 - 1
NEXP = 64
DENS = NPATCH // 2
S = NPATCH - DENS
KG = S // 2
BPG = 2
NEG = float("-inf")


def _stable_desc_ranks(v):
    n = v.shape[0]
    vj = v[:, None]
    vi = v[None, :]
    gt = (vj > vi).astype(jnp.float32)
    ioj = jax.lax.broadcasted_iota(jnp.int32, (n, n), 0)
    ioi = jax.lax.broadcasted_iota(jnp.int32, (n, n), 1)
    eq_lt = ((vj == vi) & (ioj < ioi)).astype(jnp.float32)
    return jnp.sum(gt + eq_lt, axis=0)


def _dot01(A, Bm):
    return jnp.dot(A.astype(jnp.bfloat16), Bm.astype(jnp.bfloat16),
                   preferred_element_type=jnp.float32)


def _split2(X):
    hi = X.astype(jnp.bfloat16)
    lo = (X - hi.astype(jnp.float32)).astype(jnp.bfloat16)
    return hi, lo


def _dot2(A, hi, lo):
    Ab = A.astype(jnp.bfloat16)
    return (jnp.dot(Ab, hi, preferred_element_type=jnp.float32) +
            jnp.dot(Ab, lo, preferred_element_type=jnp.float32))


def _gnn_kernel(x_ref, ca_ref, ed_ref, tok_ref, attn_ref):
    ior = jax.lax.broadcasted_iota(jnp.int32, (S, S), 0)
    ioc = jax.lax.broadcasted_iota(jnp.int32, (S, S), 1)
    iop = jax.lax.broadcasted_iota(jnp.int32, (NPATCH, NPATCH), 0)

    pre = []
    for i in range(BPG):
        xb = x_ref[i]
        ca = ca_ref[i, 0]
        ed = ed_ref[i]

        rank = _stable_desc_ranks(ca)
        P = (rank[None, :] == iop.astype(jnp.float32)).astype(jnp.float32)
        PT = (rank[:, None] == iop.T.astype(jnp.float32)).astype(jnp.float32)
        patch_x = xb[1:]
        px_hi, px_lo = _split2(patch_x)
        patch_tk = _dot2(P[:DENS], px_hi, px_lo)
        attn_s = jnp.sum(PT * ca[:, None], axis=0)
        skip_exp = jnp.dot(P[DENS:], ed[1:],
                           preferred_element_type=jnp.float32,
                           precision=jax.lax.Precision.HIGHEST)

        norm = jnp.sqrt(jnp.sum(skip_exp * skip_exp, axis=1))
        cn = skip_exp / jnp.clip(norm, 1e-12, None)[:, None]
        cnb = cn.astype(jnp.bfloat16)
        sim = jax.lax.dot_general(cnb, cnb, (((1,), (1,)), ((), ())),
                                  preferred_element_type=jnp.float32)
        sim = jnp.where(ior == ioc, NEG, sim)

        m1 = jnp.max(sim, axis=1)
        i1 = jnp.min(jnp.where(sim == m1[:, None], ioc, S), axis=1)
        hit1 = ioc == i1[:, None]
        sim2 = jnp.where(hit1, NEG, sim)
        m2 = jnp.max(sim2, axis=1)
        i2 = jnp.min(jnp.where(sim2 == m2[:, None], ioc, S), axis=1)
        hit2 = ioc == i2[:, None]
        ET = hit1 | hit2
        U = (ET | ET.T).astype(jnp.float32)
        pre.append((P, patch_tk, attn_s, U, px_hi, px_lo))

    Ub = jnp.stack([p[3] for p in pre])
    deg0 = jnp.sum(Ub, axis=1)
    C0 = Ub * (deg0[:, None, :] > deg0[:, :, None]).astype(jnp.float32)

    def cond(st):
        _, prev, cur = st
        return prev != cur

    def body(st):
        c, _, cur = st
        deg = jnp.sum(c, axis=1)
        new = c * (deg[:, None, :] > deg[:, :, None]).astype(jnp.float32)
        return new, cur, jnp.sum(new)

    Cb, _, _ = jax.lax.while_loop(
        cond, body, (C0, jnp.float32(-1.0), jnp.sum(C0)))

    eye = (ior == ioc).astype(jnp.float32)
    iog = jax.lax.broadcasted_iota(jnp.int32, (KG, S), 0)
    for i in range(BPG):
        P, patch_tk, attn_s, _, px_hi, px_lo = pre[i]
        Cf = Cb[i]
        F = Cf + eye
        node_deg = jnp.sum(F, axis=1)
        r2 = _stable_desc_ranks(node_deg)
        G = (r2[None, :] == iog.astype(jnp.float32)).astype(jnp.float32)
        GFt = _dot01(G, Cf.T) + G
        M2 = _dot01(GFt, P[DENS:])
        summaries = _dot2(M2, px_hi, px_lo)
        sca = attn_s[DENS:]
        sca_sel = jnp.sum(G * sca[None, :], axis=1)

        tok_ref[i, 0:1, :] = x_ref[i, 0:1, :]
        tok_ref[i, 1:1 + DENS, :] = patch_tk
        tok_ref[i, 1 + DENS:, :] = summaries
        attn_ref[i, 0, :DENS] = attn_s[:DENS]
        attn_ref[i, 0, DENS:] = sca_sel


def kernel(x, cls_attn, expert_distribution):
    ca3 = cls_attn.reshape(B, 1, NPATCH)
    tok, attn = pl.pallas_call(
        _gnn_kernel,
        grid=(B // BPG,),
        in_specs=[
            pl.BlockSpec((BPG, NP1, D), lambda b: (b, 0, 0)),
            pl.BlockSpec((BPG, 1, NPATCH), lambda b: (b, 0, 0)),
            pl.BlockSpec((BPG, NP1, NEXP), lambda b: (b, 0, 0)),
        ],
        out_specs=[
            pl.BlockSpec((BPG, 1 + DENS + KG, D), lambda b: (b, 0, 0)),
            pl.BlockSpec((BPG, 1, DENS + KG), lambda b: (b, 0, 0)),
        ],
        out_shape=[
            jax.ShapeDtypeStruct((B, 1 + DENS + KG, D), jnp.float32),
            jax.ShapeDtypeStruct((B, 1, DENS + KG), jnp.float32),
        ],
        compiler_params=pltpu.CompilerParams(
            dimension_semantics=("arbitrary",)),
    )(x, ca3, expert_distribution)
    return tok, attn.reshape(B, DENS + KG)


# speedup vs baseline: 1.2278x; 1.0712x over previous
import jax
import jax.numpy as jnp
from jax.experimental import pallas as pl
from jax.experimental.pallas import tpu as pltpu

B, NP1, D = 64, 577, 768
NPATCH = NP1 - 1
NEXP = 64
DENS = NPATCH // 2
S = NPATCH - DENS
KG = S // 2
BPG = 4
NEG = float("-inf")


def _stable_desc_ranks(v):
    n = v.shape[0]
    vj = v[:, None]
    vi = v[None, :]
    gt = (vj > vi).astype(jnp.float32)
    ioj = jax.lax.broadcasted_iota(jnp.int32, (n, n), 0)
    ioi = jax.lax.broadcasted_iota(jnp.int32, (n, n), 1)
    eq_lt = ((vj == vi) & (ioj < ioi)).astype(jnp.float32)
    return jnp.sum(gt + eq_lt, axis=0)


def _dot01(A, Bm):
    return jnp.dot(A.astype(jnp.bfloat16), Bm.astype(jnp.bfloat16),
                   preferred_element_type=jnp.float32)


def _split2(X):
    hi = X.astype(jnp.bfloat16)
    lo = (X - hi.astype(jnp.float32)).astype(jnp.bfloat16)
    return hi, lo


def _dot2(A, hi, lo):
    Ab = A.astype(jnp.bfloat16)
    return (jnp.dot(Ab, hi, preferred_element_type=jnp.float32) +
            jnp.dot(Ab, lo, preferred_element_type=jnp.float32))


def _gnn_kernel(x_ref, ca_ref, ed_ref, tok_ref, attn_ref):
    ior = jax.lax.broadcasted_iota(jnp.int32, (S, S), 0)
    ioc = jax.lax.broadcasted_iota(jnp.int32, (S, S), 1)
    iop = jax.lax.broadcasted_iota(jnp.int32, (NPATCH, NPATCH), 0)

    pre = []
    for i in range(BPG):
        xb = x_ref[i]
        ca = ca_ref[i, 0]
        ed = ed_ref[i]

        rank = _stable_desc_ranks(ca)
        P = (rank[None, :] == iop.astype(jnp.float32)).astype(jnp.float32)
        PT = (rank[:, None] == iop.T.astype(jnp.float32)).astype(jnp.float32)
        patch_x = xb[1:]
        px_hi, px_lo = _split2(patch_x)
        patch_tk = _dot2(P[:DENS], px_hi, px_lo)
        attn_s = jnp.sum(PT * ca[:, None], axis=0)
        skip_exp = jnp.dot(P[DENS:], ed[1:],
                           preferred_element_type=jnp.float32,
                           precision=jax.lax.Precision.HIGHEST)

        norm = jnp.sqrt(jnp.sum(skip_exp * skip_exp, axis=1))
        cn = skip_exp / jnp.clip(norm, 1e-12, None)[:, None]
        cnb = cn.astype(jnp.bfloat16)
        sim = jax.lax.dot_general(cnb, cnb, (((1,), (1,)), ((), ())),
                                  preferred_element_type=jnp.float32)
        sim = jnp.where(ior == ioc, NEG, sim)

        m1 = jnp.max(sim, axis=1)
        i1 = jnp.min(jnp.where(sim == m1[:, None], ioc, S), axis=1)
        hit1 = ioc == i1[:, None]
        sim2 = jnp.where(hit1, NEG, sim)
        m2 = jnp.max(sim2, axis=1)
        i2 = jnp.min(jnp.where(sim2 == m2[:, None], ioc, S), axis=1)
        hit2 = ioc == i2[:, None]
        ET = hit1 | hit2
        U = (ET | ET.T).astype(jnp.float32)
        pre.append((P, patch_tk, attn_s, U, px_hi, px_lo))

    Ub = jnp.stack([p[3] for p in pre])
    deg0 = jnp.sum(Ub, axis=1)
    C0 = Ub * (deg0[:, None, :] > deg0[:, :, None]).astype(jnp.float32)

    def cond(st):
        _, prev, cur = st
        return prev != cur

    def body(st):
        c, _, cur = st
        deg = jnp.sum(c, axis=1)
        new = c * (deg[:, None, :] > deg[:, :, None]).astype(jnp.float32)
        return new, cur, jnp.sum(new)

    Cb, _, _ = jax.lax.while_loop(
        cond, body, (C0, jnp.float32(-1.0), jnp.sum(C0)))

    eye = (ior == ioc).astype(jnp.float32)
    iog = jax.lax.broadcasted_iota(jnp.int32, (KG, S), 0)
    for i in range(BPG):
        P, patch_tk, attn_s, _, px_hi, px_lo = pre[i]
        Cf = Cb[i]
        F = Cf + eye
        node_deg = jnp.sum(F, axis=1)
        r2 = _stable_desc_ranks(node_deg)
        G = (r2[None, :] == iog.astype(jnp.float32)).astype(jnp.float32)
        GFt = _dot01(G, Cf.T) + G
        M2 = _dot01(GFt, P[DENS:])
        summaries = _dot2(M2, px_hi, px_lo)
        sca = attn_s[DENS:]
        sca_sel = jnp.sum(G * sca[None, :], axis=1)

        tok_ref[i, 0:1, :] = x_ref[i, 0:1, :]
        tok_ref[i, 1:1 + DENS, :] = patch_tk
        tok_ref[i, 1 + DENS:, :] = summaries
        attn_ref[i, 0, :DENS] = attn_s[:DENS]
        attn_ref[i, 0, DENS:] = sca_sel


def kernel(x, cls_attn, expert_distribution):
    ca3 = cls_attn.reshape(B, 1, NPATCH)
    tok, attn = pl.pallas_call(
        _gnn_kernel,
        grid=(B // BPG,),
        in_specs=[
            pl.BlockSpec((BPG, NP1, D), lambda b: (b, 0, 0)),
            pl.BlockSpec((BPG, 1, NPATCH), lambda b: (b, 0, 0)),
            pl.BlockSpec((BPG, NP1, NEXP), lambda b: (b, 0, 0)),
        ],
        out_specs=[
            pl.BlockSpec((BPG, 1 + DENS + KG, D), lambda b: (b, 0, 0)),
            pl.BlockSpec((BPG, 1, DENS + KG), lambda b: (b, 0, 0)),
        ],
        out_shape=[
            jax.ShapeDtypeStruct((B, 1 + DENS + KG, D), jnp.float32),
            jax.ShapeDtypeStruct((B, 1, DENS + KG), jnp.float32),
        ],
        compiler_params=pltpu.CompilerParams(
            dimension_semantics=("arbitrary",)),
    )(x, ca3, expert_distribution)
    return tok, attn.reshape(B, DENS + KG)
